# baseline (speedup 1.0000x reference)
import functools

import jax
import jax.numpy as jnp
from jax import lax
from jax.experimental import pallas as pl
from jax.experimental.pallas import tpu as pltpu
from jax.experimental.pallas import tpu_sc as plsc

D_MODEL = 2048
N_TOK = 16384
NC, NS, L = 2, 16, 16
CHUNK = 16
ROWS_C0 = 480
ROWS_C1 = 544
IDX_MAX = max(ROWS_C0, ROWS_C1)


def _sc_body(zeros_hbm, idx_hbm, out_hbm, buf0, buf1, idx_v, sem0, sem1, semi):
    bufs = (buf0, buf1)
    sems = (sem0, sem1)
    c_ax = lax.axis_index("c")
    s_ax = lax.axis_index("s")
    is0 = c_ax == 0
    n_chunks = jnp.where(is0, ROWS_C0 // CHUNK, ROWS_C1 // CHUNK)
    base = jnp.where(is0, s_ax * ROWS_C0, NS * ROWS_C0 + s_ax * ROWS_C1)
    cz0 = pltpu.make_async_copy(zeros_hbm, buf0, sem0)
    cz1 = pltpu.make_async_copy(zeros_hbm, buf1, sem1)
    ci = pltpu.make_async_copy(idx_hbm.at[pl.ds(base, IDX_MAX)], idx_v, semi)
    cz0.start()
    cz1.start()
    ci.start()
    cz0.wait()
    cz1.wait()
    ci.wait()
    row16 = lax.iota(jnp.int32, L)
    one = jnp.full((L,), 1.0, jnp.float32)
    zero = jnp.zeros((L,), jnp.float32)

    def out_slice(c):
        return out_hbm.at[pl.ds(base + c * CHUNK, CHUNK)]

    def step(g, _):
        for b in range(2):
            c = g * 2 + b

            @pl.when(c >= 2)
            def _drain():
                pltpu.make_async_copy(bufs[b], out_slice(c - 2), sems[b]).wait()
                cols_prev = idx_v[pl.ds((c - 2) * CHUNK, L)]
                plsc.store_scatter(bufs[b], [row16, cols_prev], zero)

            cols = idx_v[pl.ds(c * CHUNK, L)]
            plsc.store_scatter(bufs[b], [row16, cols], one)
            pltpu.make_async_copy(bufs[b], out_slice(c), sems[b]).start()
        return _

    lax.fori_loop(0, n_chunks // 2, step, None)
    for b in range(2):
        c_last = n_chunks - 2 + b
        pltpu.make_async_copy(bufs[b], out_slice(c_last), sems[b]).wait()


def kernel(x):
    b, s, _ = x.shape
    idx = x.reshape(N_TOK)
    zeros = jnp.zeros((CHUNK, D_MODEL), jnp.float32)
    mesh = plsc.VectorSubcoreMesh(core_axis_name="c", subcore_axis_name="s")
    k = functools.partial(
        pl.kernel,
        mesh=mesh,
        out_type=jax.ShapeDtypeStruct((N_TOK, D_MODEL), jnp.float32),
        scratch_types=[
            pltpu.VMEM((CHUNK, D_MODEL), jnp.float32),
            pltpu.VMEM((CHUNK, D_MODEL), jnp.float32),
            pltpu.VMEM((IDX_MAX,), jnp.int32),
            pltpu.SemaphoreType.DMA,
            pltpu.SemaphoreType.DMA,
            pltpu.SemaphoreType.DMA,
        ],
        compiler_params=pltpu.CompilerParams(needs_layout_passes=False),
    )(_sc_body)
    out = k(zeros, idx)
    return (out.reshape(b, s, D_MODEL),)


# speedup vs baseline: 1.0034x; 1.0034x over previous
import functools

import jax
import jax.numpy as jnp
from jax import lax
from jax.experimental import pallas as pl
from jax.experimental.pallas import tpu as pltpu
from jax.experimental.pallas import tpu_sc as plsc

D_MODEL = 2048
N_TOK = 16384
NC, NS, L = 2, 16, 16
CHUNK = 16
ROWS_C0 = 480
ROWS_C1 = 544
IDX_MAX = max(ROWS_C0, ROWS_C1)


def _sc_body(zeros_hbm, idx_hbm, out_hbm, buf0, buf1, idx_v, sem0, sem1):
    bufs = (buf0, buf1)
    sems = (sem0, sem1)
    c_ax = lax.axis_index("c")
    s_ax = lax.axis_index("s")
    is0 = c_ax == 0
    n_chunks = jnp.where(is0, ROWS_C0 // CHUNK, ROWS_C1 // CHUNK)
    base = jnp.where(is0, s_ax * ROWS_C0, NS * ROWS_C0 + s_ax * ROWS_C1)
    pltpu.sync_copy(zeros_hbm, buf0)
    pltpu.sync_copy(zeros_hbm, buf1)
    pltpu.sync_copy(idx_hbm.at[pl.ds(base, IDX_MAX)], idx_v)
    row16 = lax.iota(jnp.int32, L)
    one = jnp.full((L,), 1.0, jnp.float32)
    zero = jnp.zeros((L,), jnp.float32)

    def out_slice(c):
        return out_hbm.at[pl.ds(base + c * CHUNK, CHUNK)]

    def step(g, _):
        for b in range(2):
            c = g * 2 + b

            @pl.when(c >= 2)
            def _drain():
                pltpu.make_async_copy(bufs[b], out_slice(c - 2), sems[b]).wait()
                cols_prev = idx_v[pl.ds((c - 2) * CHUNK, L)]
                plsc.store_scatter(bufs[b], [row16, cols_prev], zero)

            cols = idx_v[pl.ds(c * CHUNK, L)]
            plsc.store_scatter(bufs[b], [row16, cols], one)
            pltpu.make_async_copy(bufs[b], out_slice(c), sems[b]).start()
        return _

    lax.fori_loop(0, n_chunks // 2, step, None)
    for b in range(2):
        c_last = n_chunks - 2 + b
        pltpu.make_async_copy(bufs[b], out_slice(c_last), sems[b]).wait()


def kernel(x):
    b, s, _ = x.shape
    idx = x.reshape(N_TOK)
    zeros = jnp.zeros((CHUNK, D_MODEL), jnp.float32)
    mesh = plsc.VectorSubcoreMesh(core_axis_name="c", subcore_axis_name="s")
    k = functools.partial(
        pl.kernel,
        mesh=mesh,
        out_type=jax.ShapeDtypeStruct((N_TOK, D_MODEL), jnp.float32),
        scratch_types=[
            pltpu.VMEM((CHUNK, D_MODEL), jnp.float32),
            pltpu.VMEM((CHUNK, D_MODEL), jnp.float32),
            pltpu.VMEM((IDX_MAX,), jnp.int32),
            pltpu.SemaphoreType.DMA,
            pltpu.SemaphoreType.DMA,
        ],
        compiler_params=pltpu.CompilerParams(needs_layout_passes=False),
    )(_sc_body)
    out = k(zeros, idx)
    return (out.reshape(b, s, D_MODEL),)


# speedup vs baseline: 1.0047x; 1.0013x over previous
import functools

import jax
import jax.numpy as jnp
from jax import lax
from jax.experimental import pallas as pl
from jax.experimental.pallas import tpu as pltpu
from jax.experimental.pallas import tpu_sc as plsc

D_MODEL = 2048
N_TOK = 16384
NC, NS, L = 2, 16, 16
CHUNK = 16
ROWS_C0 = 480
ROWS_C1 = 544
IDX_MAX = max(ROWS_C0, ROWS_C1)


def _sc_body(zeros_hbm, idx_hbm, out_hbm, buf0, buf1, idx_v, sem0, sem1):
    bufs = (buf0, buf1)
    sems = (sem0, sem1)
    c_ax = lax.axis_index("c")
    s_ax = lax.axis_index("s")
    is0 = c_ax == 0
    n_chunks = jnp.where(is0, ROWS_C0 // CHUNK, ROWS_C1 // CHUNK)
    base = jnp.where(is0, s_ax * ROWS_C0, NS * ROWS_C0 + s_ax * ROWS_C1)
    pltpu.sync_copy(zeros_hbm, buf0)
    pltpu.sync_copy(zeros_hbm, buf1)
    pltpu.sync_copy(idx_hbm.at[pl.ds(base, IDX_MAX)], idx_v)
    row16 = lax.iota(jnp.int32, L)
    one = jnp.full((L,), 1.0, jnp.float32)
    zero = jnp.zeros((L,), jnp.float32)

    def out_slice(c):
        return out_hbm.at[pl.ds(base + c * CHUNK, CHUNK)]

    def step(g, _):
        for b in range(2):
            c = g * 2 + b

            @pl.when(c >= 2)
            def _drain():
                pltpu.make_async_copy(bufs[b], out_slice(c - 2), sems[b]).wait()
                cols_prev = idx_v[pl.ds((c - 2) * CHUNK, L)]
                plsc.store_scatter(bufs[b], [row16, cols_prev], zero)

            cols = idx_v[pl.ds(c * CHUNK, L)]
            plsc.store_scatter(bufs[b], [row16, cols], one)
            pltpu.make_async_copy(bufs[b], out_slice(c), sems[b]).start()
        return _

    lax.fori_loop(0, n_chunks // 2, step, None)
    for b in range(2):
        c_last = n_chunks - 2 + b
        pltpu.make_async_copy(bufs[b], out_slice(c_last), sems[b]).wait()


def kernel(x):
    b, s, _ = x.shape
    idx = x.reshape(N_TOK)
    zeros = jnp.zeros((CHUNK, D_MODEL), jnp.float32)
    mesh = plsc.VectorSubcoreMesh(core_axis_name="c", subcore_axis_name="s")
    k = functools.partial(
        pl.kernel,
        mesh=mesh,
        out_type=jax.ShapeDtypeStruct((N_TOK, D_MODEL), jnp.float32),
        scratch_types=[
            pltpu.VMEM((CHUNK, D_MODEL), jnp.float32),
            pltpu.VMEM((CHUNK, D_MODEL), jnp.float32),
            pltpu.VMEM((IDX_MAX,), jnp.int32),
            pltpu.SemaphoreType.DMA,
            pltpu.SemaphoreType.DMA,
        ],
        compiler_params=pltpu.CompilerParams(
            needs_layout_passes=False,
            disable_bounds_checks=True,
            disable_semaphore_checks=True,
        ),
    )(_sc_body)
    out = k(zeros, idx)
    return (out.reshape(b, s, D_MODEL),)
